# Initial kernel scaffold; baseline (speedup 1.0000x reference)
import jax
import jax.numpy as jnp
from jax.experimental import pallas as pl


def kernel(var_feats, cstr_feats, edge_index, edge_attr, batch_el, config_batch, W_var, b_var, W_cstr, b_cstr, Wrn, brn, Wqn, Wrc, brc, Wqc, W_out, b_out, Wc1, bc1, Wc2, bc2, Wh, bh, Wo, bo):
    raise NotImplementedError("write your pallas kernel here")


# speedup vs baseline: 1.0000x; 1.0000x over previous
import jax
import jax.numpy as jnp
from jax.experimental import pallas as pl

N_VAR = 100000
N_CSTR = 100000
B = 256
L = 4


def kernel(var_feats, cstr_feats, edge_index, edge_attr, batch_el, config_batch,
           W_var, b_var, W_cstr, b_cstr,
           Wrn, brn, Wqn, Wrc, brc, Wqc,
           W_out, b_out, Wc1, bc1, Wc2, bc2, Wh, bh, Wo, bo):
    xv = jax.nn.relu(var_feats @ W_var + b_var)
    xc = jax.nn.relu(cstr_feats @ W_cstr + b_cstr)
    ew = edge_attr[:, None]
    src = edge_index[0]
    dst = edge_index[1]
    for l in range(L):
        msg_v = xc[src] * ew
        agg_v = jax.ops.segment_sum(msg_v, dst, num_segments=N_VAR)
        xv_new = agg_v @ Wrn[l] + brn[l] + xv @ Wqn[l]
        msg_c = xv[dst] * ew
        agg_c = jax.ops.segment_sum(msg_c, src, num_segments=N_CSTR)
        xc_new = agg_c @ Wrc[l] + brc[l] + xc @ Wqc[l]
        xv = jax.nn.relu(xv_new)
        xc = jax.nn.relu(xc_new)
    sums = jax.ops.segment_sum(xv, batch_el, num_segments=B)
    cnts = jax.ops.segment_sum(jnp.ones((N_VAR, 1), xv.dtype), batch_el, num_segments=B)
    pooled = sums / jnp.maximum(cnts, 1.0)
    g = jax.nn.relu(pooled @ W_out + b_out)
    h = jax.nn.relu(jnp.concatenate([g, jnp.zeros_like(g)], axis=-1) @ Wh + bh)
    reg = h @ Wo + bo
    return jnp.concatenate([reg[:, 0:1], jnp.exp(reg[:, 1:2])], axis=-1)


# speedup vs baseline: 29.7026x; 29.7016x over previous
import functools

import jax
import jax.numpy as jnp
from jax import lax
from jax.experimental import pallas as pl
from jax.experimental.pallas import tpu as pltpu
from jax.experimental.pallas import tpu_sc as plsc

NV = 100000
NCS = 100000
E = 6400000
B = 256
H = 8
L = 4

NCORES = 2
NSUB = 16
NW = NCORES * NSUB
CHUNK = 2048
NCHUNK = E // CHUNK
KMAX = -(-NCHUNK // NW)
RW = NV // NW
PB = 25
PR = RW // PB

_f32 = jnp.float32


def _agg_body(xv_hbm, xc_hbm, src3, dst3, w2, zeros_hbm,
              aggv_out, aggc_out,
              src_v, dst_v, w_v, rows_a, rows_b, spm,
              lsem, gsem, ssem):
    c = lax.axis_index("c")
    s = lax.axis_index("s")
    wid = s * NCORES + c
    row0 = s * RW

    iota = lax.iota(jnp.int32, 16)
    pat_e = lax.shift_right_logical(iota, 3)
    pat_f = jnp.bitwise_and(iota, 7)

    def _scale(rows, base):
        def body(i, _):
            for u in range(8):
                v = i * 8 + u
                e_loc = pat_e + 2 * v
                rv = plsc.load_gather(rows, [e_loc, pat_f])
                wv = plsc.load_gather(w_v, [e_loc + base])
                plsc.store_scatter(rows, [e_loc, pat_f], rv * wv)
            return 0
        lax.fori_loop(0, 64, body, 0)

    for d in range(2):
        x_hbm = xc_hbm if d == 0 else xv_hbm
        g_idx = src_v if d == 0 else dst_v
        o_idx = dst_v if d == 0 else src_v
        out = aggv_out if d == 0 else aggc_out

        pltpu.sync_copy(zeros_hbm.at[pl.ds(row0, RW)],
                        spm.at[pl.ds(row0, RW)])
        plsc.subcore_barrier()

        def chunk_body(k, _):
            ch = wid + NW * k

            @pl.when(ch < NCHUNK)
            def _():
                l0 = pltpu.async_copy(src3.at[ch], src_v, lsem)
                l1 = pltpu.async_copy(dst3.at[ch], dst_v, lsem)
                l2 = pltpu.async_copy(w2.at[ch], w_v, lsem)
                l0.wait(); l1.wait(); l2.wait()

                gda = [pltpu.async_copy(x_hbm.at[g_idx.at[j]],
                                        rows_a.at[pl.ds(j * 128, 128)], gsem)
                       for j in range(8)]
                for dd in gda:
                    dd.wait()
                _scale(rows_a, 0)
                sda = [pltpu.async_copy(rows_a.at[pl.ds(j * 128, 128)],
                                        spm.at[o_idx.at[j]], ssem, add=True)
                       for j in range(8)]

                gdb = [pltpu.async_copy(x_hbm.at[g_idx.at[8 + j]],
                                        rows_b.at[pl.ds(j * 128, 128)], gsem)
                       for j in range(8)]
                for dd in gdb:
                    dd.wait()
                _scale(rows_b, 1024)
                sdb = [pltpu.async_copy(rows_b.at[pl.ds(j * 128, 128)],
                                        spm.at[o_idx.at[8 + j]], ssem,
                                        add=True)
                       for j in range(8)]

                for dd in sda:
                    dd.wait()
                for dd in sdb:
                    dd.wait()
            return 0

        lax.fori_loop(0, KMAX, chunk_body, 0)

        plsc.subcore_barrier()
        pltpu.sync_copy(spm.at[pl.ds(row0, RW)],
                        out.at[c].at[pl.ds(row0, RW)])
        plsc.subcore_barrier()


def _pool_body(xv_hbm, bid3, ones_hbm, zeros_hbm,
               sums_out, cnts_out,
               xrows, bid_v, ones_v, spm_s, spm_n, sem):
    c = lax.axis_index("c")
    s = lax.axis_index("s")
    wid = s * NCORES + c

    @pl.when(s == 0)
    def _():
        pltpu.sync_copy(zeros_hbm, spm_s)
        pltpu.sync_copy(zeros_hbm, spm_n)

    plsc.subcore_barrier()

    pltpu.sync_copy(xv_hbm.at[pl.ds(wid * RW, RW)], xrows)
    pltpu.sync_copy(bid3.at[wid], bid_v)
    pltpu.sync_copy(ones_hbm, ones_v)

    descs = []
    for j in range(PB):
        descs.append(pltpu.async_copy(xrows.at[pl.ds(j * PR, PR)],
                                      spm_s.at[bid_v.at[j]], sem, add=True))
        descs.append(pltpu.async_copy(ones_v,
                                      spm_n.at[bid_v.at[j]], sem, add=True))
    for d in descs:
        d.wait()

    plsc.subcore_barrier()

    @pl.when(s == 0)
    def _():
        pltpu.sync_copy(spm_s, sums_out.at[c])
        pltpu.sync_copy(spm_n, cnts_out.at[c])


def _embed_tc(vf, cf, kv, kc, bv, bc, xv_out, xc_out):
    xv_out[...] = jnp.maximum(
        jnp.dot(vf[...], kv[...], preferred_element_type=_f32) + bv[...], 0.0)
    xc_out[...] = jnp.maximum(
        jnp.dot(cf[...], kc[...], preferred_element_type=_f32) + bc[...], 0.0)


def _update_tc(aggv2, aggc2, xv, xc, krn, kqn, krc, kqc, bn, bc_,
               xv_out, xc_out):
    aggv = aggv2[0] + aggv2[1]
    aggc = aggc2[0] + aggc2[1]
    xv_out[...] = jnp.maximum(
        jnp.dot(aggv, krn[...], preferred_element_type=_f32) + bn[...]
        + jnp.dot(xv[...], kqn[...], preferred_element_type=_f32), 0.0)
    xc_out[...] = jnp.maximum(
        jnp.dot(aggc, krc[...], preferred_element_type=_f32) + bc_[...]
        + jnp.dot(xc[...], kqc[...], preferred_element_type=_f32), 0.0)


def _head_tc(s2, c2, kout, bout, kh, bh_, ko, bo_, out_ref):
    pooled = (s2[0] + s2[1]) / jnp.maximum(c2[0] + c2[1], 1.0)
    g = jnp.maximum(
        jnp.dot(pooled, kout[...], preferred_element_type=_f32) + bout[...], 0.0)
    h = jnp.maximum(
        jnp.dot(g, kh[...], preferred_element_type=_f32) + bh_[...], 0.0)
    reg = jnp.dot(h, ko[...], preferred_element_type=_f32) + bo_[...]
    col = lax.broadcasted_iota(jnp.int32, (16, 32), 1)
    out_ref[...] = jnp.where(col % 2 == 0, reg, jnp.exp(reg))


def _kron16(w):
    return jnp.kron(jnp.eye(16, dtype=_f32), w.astype(_f32))


def _tile16(b):
    return jnp.tile(b.astype(_f32), 16)[None, :]


def kernel(var_feats, cstr_feats, edge_index, edge_attr, batch_el,
           config_batch,
           W_var, b_var, W_cstr, b_cstr,
           Wrn, brn, Wqn, Wrc, brc, Wqc,
           W_out, b_out, Wc1, bc1, Wc2, bc2, Wh, bh, Wo, bo):
    src3 = edge_index[0].reshape(NCHUNK, 16, 128)
    dst3 = edge_index[1].reshape(NCHUNK, 16, 128)
    w2 = edge_attr.reshape(NCHUNK, CHUNK)
    bid3 = batch_el.reshape(NW, PB, PR)
    zeros_big = jnp.zeros((NV, H), _f32)
    zeros_b = jnp.zeros((B, H), _f32)
    ones_s = jnp.ones((PR, H), _f32)

    mesh = plsc.VectorSubcoreMesh(core_axis_name="c", subcore_axis_name="s")
    sc_params = pltpu.CompilerParams(use_tc_tiling_on_sc=False,
                                     needs_layout_passes=False)

    agg = pl.kernel(
        _agg_body,
        out_type=[jax.ShapeDtypeStruct((NCORES, NV, H), _f32),
                  jax.ShapeDtypeStruct((NCORES, NCS, H), _f32)],
        mesh=mesh,
        scratch_types=[
            pltpu.VMEM((16, 128), jnp.int32),
            pltpu.VMEM((16, 128), jnp.int32),
            pltpu.VMEM((CHUNK,), _f32),
            pltpu.VMEM((CHUNK // 2, H), _f32),
            pltpu.VMEM((CHUNK // 2, H), _f32),
            pltpu.VMEM_SHARED((NV, H), _f32),
            pltpu.SemaphoreType.DMA,
            pltpu.SemaphoreType.DMA,
            pltpu.SemaphoreType.DMA,
        ],
        compiler_params=sc_params,
    )

    pool = pl.kernel(
        _pool_body,
        out_type=[jax.ShapeDtypeStruct((NCORES, B, H), _f32),
                  jax.ShapeDtypeStruct((NCORES, B, H), _f32)],
        mesh=mesh,
        scratch_types=[
            pltpu.VMEM((RW, H), _f32),
            pltpu.VMEM((PB, PR), jnp.int32),
            pltpu.VMEM((PR, H), _f32),
            pltpu.VMEM_SHARED((B, H), _f32),
            pltpu.VMEM_SHARED((B, H), _f32),
            pltpu.SemaphoreType.DMA,
        ],
        compiler_params=sc_params,
    )

    embed = pl.pallas_call(
        _embed_tc,
        out_shape=[jax.ShapeDtypeStruct((NV // 16, 128), _f32),
                   jax.ShapeDtypeStruct((NCS // 16, 128), _f32)],
    )
    update = pl.pallas_call(
        _update_tc,
        out_shape=[jax.ShapeDtypeStruct((NV // 16, 128), _f32),
                   jax.ShapeDtypeStruct((NCS // 16, 128), _f32)],
    )
    head = pl.pallas_call(
        _head_tc,
        out_shape=jax.ShapeDtypeStruct((16, 32), _f32),
    )

    xv_r, xc_r = embed(
        var_feats.reshape(NV // 16, 144),
        cstr_feats.reshape(NCS // 16, 16),
        _kron16(W_var), _kron16(W_cstr),
        _tile16(b_var), _tile16(b_cstr))

    for l in range(L):
        aggv2, aggc2 = agg(xv_r.reshape(NV, H), xc_r.reshape(NCS, H),
                           src3, dst3, w2, zeros_big)
        xv_r, xc_r = update(
            aggv2.reshape(NCORES, NV // 16, 128),
            aggc2.reshape(NCORES, NCS // 16, 128),
            xv_r, xc_r,
            _kron16(Wrn[l]), _kron16(Wqn[l]),
            _kron16(Wrc[l]), _kron16(Wqc[l]),
            _tile16(brn[l]), _tile16(brc[l]))

    sums2, cnts2 = pool(xv_r.reshape(NV, H), bid3, ones_s, zeros_b)

    out_r = head(sums2.reshape(NCORES, 16, 128),
                 cnts2.reshape(NCORES, 16, 128),
                 _kron16(W_out), _tile16(b_out),
                 _kron16(Wh[:H]), _tile16(bh),
                 _kron16(Wo), jnp.tile(bo.astype(_f32), 16)[None, :])
    return out_r.reshape(B, 2)
